# baseline (speedup 1.0000x reference)
import jax
import jax.numpy as jnp
from jax import lax
from jax.experimental import pallas as pl
from jax.experimental.pallas import tpu as pltpu
from jax.experimental.pallas import tpu_sc as plsc

_B = 4096
_F = 26
_V = 100000
_E = 32
_DENSE = 64
_OUT = _F * _E + _DENSE

_FH = _F // 2
_NC = 2
_NS = 16
_NW = _NC * _NS
_ITEMS = _B * _FH
_IPW = _ITEMS // _NW
_CHUNK_ITEMS = 416
_NCHUNK = _IPW // _CHUNK_ITEMS
_STREAM_IDX = 128
_NSTREAM = _CHUNK_ITEMS * _E // _STREAM_IDX
_LANES = 16


def _sc_gather_body(idx_hbm, tab_hbm, emb_hbm, idx_v, ebuf, rows, sem):
    wid = lax.axis_index("s") * _NC + lax.axis_index("c")

    pltpu.sync_copy(idx_hbm.at[wid], idx_v)

    def _off_body(t, carry):
        r = t // (128 // _LANES)
        c = t % (128 // _LANES)
        p = r * 128 + c * _LANES + lax.iota(jnp.int32, _LANES)
        f = lax.rem(p, jnp.int32(_FH))
        cur = idx_v[r, pl.ds(c * _LANES, _LANES)]
        idx_v[r, pl.ds(c * _LANES, _LANES)] = cur + f * jnp.int32(_E * _V)
        return carry

    lax.fori_loop(0, _FH * (128 // _LANES), _off_body, 0)

    iota16 = lax.iota(jnp.int32, _LANES)
    e_lo = iota16 * jnp.int32(_V)
    e_hi = e_lo + jnp.int32(_LANES * _V)

    for c in range(_NCHUNK):
        def _build(g, carry):
            p0 = c * _CHUNK_ITEMS + g * _LANES
            base = idx_v[p0 // 128, pl.ds(p0 % 128, _LANES)]
            d0 = g * (_LANES * _E)
            for k in range(2 * _LANES):
                val = base[k // 2] + (e_lo if k % 2 == 0 else e_hi)
                ebuf[pl.ds(d0 + k * _LANES, _LANES)] = val
            return carry

        lax.fori_loop(0, _CHUNK_ITEMS // _LANES, _build, 0)

        def _fire(s, carry):
            pltpu.make_async_copy(
                tab_hbm.at[ebuf.at[pl.ds(s * _STREAM_IDX, _STREAM_IDX)]],
                rows.at[pl.ds(s * _STREAM_IDX, _STREAM_IDX)],
                sem,
            ).start()
            return carry

        lax.fori_loop(0, _NSTREAM, _fire, 0)

        def _drain(s, carry):
            pltpu.make_async_copy(
                tab_hbm.at[ebuf.at[pl.ds(s * _STREAM_IDX, _STREAM_IDX)]],
                rows.at[pl.ds(s * _STREAM_IDX, _STREAM_IDX)],
                sem,
            ).wait()
            return carry

        lax.fori_loop(0, _NSTREAM, _drain, 0)

        base_out = wid * _IPW * _E + c * _CHUNK_ITEMS * _E
        pltpu.sync_copy(rows, emb_hbm.at[pl.ds(base_out, _CHUNK_ITEMS * _E)])


def _make_half_gather():
    mesh = plsc.VectorSubcoreMesh(core_axis_name="c", subcore_axis_name="s")
    return pl.kernel(
        _sc_gather_body,
        mesh=mesh,
        out_type=jax.ShapeDtypeStruct((_ITEMS * _E,), jnp.float32),
        scratch_types=[
            pltpu.VMEM((_FH, 128), jnp.int32),
            pltpu.VMEM((_CHUNK_ITEMS * _E,), jnp.int32),
            pltpu.VMEM((_CHUNK_ITEMS * _E,), jnp.float32),
            pltpu.SemaphoreType.DMA,
        ],
        compiler_params=pltpu.CompilerParams(use_tc_tiling_on_sc=False),
    )


@jax.jit
def _sc_gather2(idx1, idx2, tab1, tab2):
    f = _make_half_gather()
    return f(idx1, tab1), f(idx2, tab2)


def _bn_body(e1_ref, e2_ref, dense_ref, gamma_ref, beta_ref, out_ref):
    x = jnp.concatenate([e1_ref[...], e2_ref[...], dense_ref[...]], axis=-1)
    s1 = jnp.sum(x, axis=0, keepdims=True)
    s2 = jnp.sum(x * x, axis=0, keepdims=True)
    mean = s1 * (1.0 / _B)
    var = s2 * (1.0 / _B) - mean * mean
    inv = lax.rsqrt(var + 1e-3)
    out_ref[...] = (x - mean) * (inv * gamma_ref[...]) + beta_ref[...]


def _bn(e1, e2, dense, gamma2, beta2):
    return pl.pallas_call(
        _bn_body,
        out_shape=jax.ShapeDtypeStruct((_B, _OUT), jnp.float32),
    )(e1, e2, dense, gamma2, beta2)


def kernel(indices, dense, tables, gamma, beta):
    idx1 = indices[:, :_FH].reshape(_NW, _FH, 128)
    idx2 = indices[:, _FH:].reshape(_NW, _FH, 128)
    tab1 = jnp.transpose(tables[:_FH], (0, 2, 1)).reshape(-1)
    tab2 = jnp.transpose(tables[_FH:], (0, 2, 1)).reshape(-1)
    h1, h2 = _sc_gather2(idx1, idx2, tab1, tab2)
    e1 = h1.reshape(_B, _FH * _E)
    e2 = h2.reshape(_B, _FH * _E)
    gamma2 = gamma.reshape(1, _OUT)
    beta2 = beta.reshape(1, _OUT)
    return _bn(e1, e2, dense, gamma2, beta2)
